# baseline (speedup 1.0000x reference)
import jax
import jax.numpy as jnp
from jax import lax
from jax.experimental import pallas as pl
from jax.experimental.pallas import tpu as pltpu

N = 1500
D = 2
HID = 32
K = 4
NW = N // K

_OFF_FEATS = 2
_OFF_ADJ = _OFF_FEATS + N * D
_OFF_CAND = _OFF_ADJ + N * N
_OFF_MASK = _OFF_CAND + N
_ROW = _OFF_MASK + N


def _fused(adj_ref, feats_ref, cand_ref, mask_ref, act_ref,
           w00, b00, w01, b01, w02, b02,
           w10, b10, w11, b11, w12, b12,
           aw0, ab0, aw1, ab1, aw2, ab2,
           cw0, cb0, cw1, cb1, pm,
           out_ref):
    f32 = jnp.float32
    bf16 = jnp.bfloat16
    wi = adj_ref[0].astype(jnp.int32)
    planes = [((wi >> k) & 1).astype(f32) for k in range(K)]
    feats = feats_ref[0]

    def dot(a, b):
        return jnp.dot(a, b, preferred_element_type=f32)

    def agg(h):
        cs = [dot(p, h) for p in planes]
        return jnp.stack(cs, axis=1).reshape(N, h.shape[1])

    pooled = agg(feats) + feats
    t = jnp.maximum(dot(pooled, w00[...]) + b00[...], 0.0)
    t = jnp.maximum(dot(t, w01[...]) + b01[...], 0.0)
    t = dot(t, w02[...]) + b02[...]
    h = jnp.maximum(t, 0.0)

    pooled = agg(h) + h
    t = jnp.maximum(dot(pooled, w10[...]) + b10[...], 0.0)
    t = jnp.maximum(dot(t, w11[...]) + b11[...], 0.0)
    t = dot(t, w12[...]) + b12[...]
    h = jnp.maximum(t, 0.0)

    grow = jnp.full((1, N), 1.0 / N, dtype=f32)
    g = dot(grow, h)

    cand = cand_ref[0]
    cols = lax.broadcasted_iota(jnp.int32, (N, N), 1)
    onehot = (cols == cand).astype(bf16)
    h_hi = h.astype(bf16)
    h_lo = (h - h_hi.astype(f32)).astype(bf16)
    job = (jnp.dot(onehot, h_hi, preferred_element_type=f32)
           + jnp.dot(onehot, h_lo, preferred_element_type=f32))

    cat = jnp.concatenate(
        [job,
         jnp.broadcast_to(g, (N, HID)),
         jnp.broadcast_to(pm[...], (N, HID))], axis=1)

    a = jnp.tanh(dot(cat, aw0[...]) + ab0[...])
    a = jnp.tanh(dot(a, aw1[...]) + ab1[...])
    s = dot(a, aw2[...]) + ab2[...]
    scores = s * 10.0
    mask = mask_ref[0]
    scores = jnp.where(mask != 0.0, -jnp.inf, scores)

    m = jnp.max(scores, axis=0, keepdims=True)
    e = jnp.exp(scores - m)
    logits = e / jnp.sum(e, axis=0, keepdims=True)

    m2 = jnp.max(logits, axis=0, keepdims=True)
    ls2 = m2 + jnp.log(jnp.sum(jnp.exp(logits - m2), axis=0, keepdims=True))
    logp_all = logits - ls2
    p = jnp.exp(logp_all)
    ent = -jnp.sum(p * logp_all, axis=0, keepdims=True)

    ai = act_ref[0, 0, 0]
    rows = lax.broadcasted_iota(jnp.int32, (N, 1), 0)
    logp = jnp.sum(jnp.where(rows == ai, logp_all, 0.0), axis=0, keepdims=True)

    c = jnp.tanh(dot(g, cw0[...]) + cb0[...])
    v = dot(c, cw1[...]) + cb1[...]

    lanes = lax.broadcasted_iota(jnp.int32, (1, 1, 128), 2)
    out = jnp.where(lanes == 0, logp[0, 0],
          jnp.where(lanes == 1, ent[0, 0],
          jnp.where(lanes == 2, v[0, 0], 0.0)))
    out_ref[...] = out


def kernel(x, action, enc_W0_0, enc_b0_0, enc_W0_1, enc_b0_1, enc_W0_2, enc_b0_2,
           enc_W1_0, enc_b1_0, enc_W1_1, enc_b1_1, enc_W1_2, enc_b1_2,
           actor_W0, actor_b0, actor_W1, actor_b1, actor_W2, actor_b2,
           critic_W0, critic_b0, critic_W1, critic_b1, pooled_machine):
    B = x.shape[0]
    f32 = jnp.float32
    feats = x[:, _OFF_FEATS:_OFF_ADJ].reshape(B, N, D)
    af = x[:, _OFF_ADJ:_OFF_CAND].reshape(B, NW, K, N)
    adj = (af[:, :, 0, :] + 2.0 * af[:, :, 1, :] + 4.0 * af[:, :, 2, :]
           + 8.0 * af[:, :, 3, :]).astype(jnp.uint8)
    cand = x[:, _OFF_CAND:_OFF_MASK].astype(jnp.int32).reshape(B, N, 1)
    mask = x[:, _OFF_MASK:_ROW].reshape(B, N, 1)
    act3 = action.astype(jnp.int32).reshape(B, 1, 1)

    def row2(v):
        return v.reshape(1, -1).astype(f32)

    per_sample = lambda bs: pl.BlockSpec(bs, lambda b: (b,) + (0,) * (len(bs) - 1))
    shared = lambda arr: pl.BlockSpec(arr.shape, lambda b: (0,) * arr.ndim)

    weights = [enc_W0_0, row2(enc_b0_0), enc_W0_1, row2(enc_b0_1), enc_W0_2, row2(enc_b0_2),
               enc_W1_0, row2(enc_b1_0), enc_W1_1, row2(enc_b1_1), enc_W1_2, row2(enc_b1_2),
               actor_W0, row2(actor_b0), actor_W1, row2(actor_b1), actor_W2, row2(actor_b2),
               critic_W0, row2(critic_b0), critic_W1, row2(critic_b1), row2(pooled_machine)]

    in_specs = [per_sample((1, NW, N)),
                per_sample((1, N, D)),
                per_sample((1, N, 1)), per_sample((1, N, 1)),
                per_sample((1, 1, 1))] + [shared(w) for w in weights]

    out = pl.pallas_call(
        _fused,
        grid=(B,),
        in_specs=in_specs,
        out_specs=pl.BlockSpec((1, 1, 128), lambda b: (b, 0, 0)),
        out_shape=jax.ShapeDtypeStruct((B, 1, 128), f32),
        compiler_params=pltpu.CompilerParams(
            dimension_semantics=("parallel",),
            vmem_limit_bytes=120 * 1024 * 1024),
    )(adj, feats, cand, mask, act3, *weights)

    return action, out[:, 0, 0], out[:, 0, 1], out[:, 0, 2:3]


# speedup vs baseline: 5.4941x; 5.4941x over previous
import jax
import jax.numpy as jnp
from jax import lax
from jax.experimental import pallas as pl
from jax.experimental.pallas import tpu as pltpu

N = 1500
D = 2
HID = 32

_OFF_FEATS = 2
_OFF_ADJ = _OFF_FEATS + N * D
_OFF_CAND = _OFF_ADJ + N * N
_OFF_MASK = _OFF_CAND + N
_ROW = _OFF_MASK + N


def _fused(adj_ref, feats_ref, cand_ref, mask_ref, act_ref,
           w00, b00, w01, b01, w02, b02,
           w10, b10, w11, b11, w12, b12,
           aw0, ab0, aw1, ab1, aw2, ab2,
           cw0, cb0, cw1, cb1, pm,
           out_ref):
    f32 = jnp.float32
    bf16 = jnp.bfloat16
    rows = [adj_ref[0, 0, _OFF_ADJ + i * N:_OFF_ADJ + (i + 1) * N].reshape(1, N)
            for i in range(N)]
    adj = jnp.concatenate(rows, axis=0)
    feats = feats_ref[0]

    def dot(a, b):
        return jnp.dot(a, b, preferred_element_type=f32)

    pooled = dot(adj, feats) + feats
    t = jnp.maximum(dot(pooled, w00[...]) + b00[...], 0.0)
    t = jnp.maximum(dot(t, w01[...]) + b01[...], 0.0)
    t = dot(t, w02[...]) + b02[...]
    h = jnp.maximum(t, 0.0)

    pooled = dot(adj, h) + h
    t = jnp.maximum(dot(pooled, w10[...]) + b10[...], 0.0)
    t = jnp.maximum(dot(t, w11[...]) + b11[...], 0.0)
    t = dot(t, w12[...]) + b12[...]
    h = jnp.maximum(t, 0.0)

    grow = jnp.full((1, N), 1.0 / N, dtype=f32)
    g = dot(grow, h)

    cand = cand_ref[0]
    cols = lax.broadcasted_iota(jnp.int32, (N, N), 1)
    onehot = (cols == cand).astype(bf16)
    h_hi = h.astype(bf16)
    h_lo = (h - h_hi.astype(f32)).astype(bf16)
    job = (jnp.dot(onehot, h_hi, preferred_element_type=f32)
           + jnp.dot(onehot, h_lo, preferred_element_type=f32))

    cat = jnp.concatenate(
        [job,
         jnp.broadcast_to(g, (N, HID)),
         jnp.broadcast_to(pm[...], (N, HID))], axis=1)

    a = jnp.tanh(dot(cat, aw0[...]) + ab0[...])
    a = jnp.tanh(dot(a, aw1[...]) + ab1[...])
    s = dot(a, aw2[...]) + ab2[...]
    scores = s * 10.0
    mask = mask_ref[0]
    scores = jnp.where(mask != 0.0, -jnp.inf, scores)

    m = jnp.max(scores, axis=0, keepdims=True)
    e = jnp.exp(scores - m)
    logits = e / jnp.sum(e, axis=0, keepdims=True)

    m2 = jnp.max(logits, axis=0, keepdims=True)
    ls2 = m2 + jnp.log(jnp.sum(jnp.exp(logits - m2), axis=0, keepdims=True))
    logp_all = logits - ls2
    p = jnp.exp(logp_all)
    ent = -jnp.sum(p * logp_all, axis=0, keepdims=True)

    ai = act_ref[0, 0, 0]
    rows = lax.broadcasted_iota(jnp.int32, (N, 1), 0)
    logp = jnp.sum(jnp.where(rows == ai, logp_all, 0.0), axis=0, keepdims=True)

    c = jnp.tanh(dot(g, cw0[...]) + cb0[...])
    v = dot(c, cw1[...]) + cb1[...]

    lanes = lax.broadcasted_iota(jnp.int32, (1, 1, 128), 2)
    out = jnp.where(lanes == 0, logp[0, 0],
          jnp.where(lanes == 1, ent[0, 0],
          jnp.where(lanes == 2, v[0, 0], 0.0)))
    out_ref[...] = out


def kernel(x, action, enc_W0_0, enc_b0_0, enc_W0_1, enc_b0_1, enc_W0_2, enc_b0_2,
           enc_W1_0, enc_b1_0, enc_W1_1, enc_b1_1, enc_W1_2, enc_b1_2,
           actor_W0, actor_b0, actor_W1, actor_b1, actor_W2, actor_b2,
           critic_W0, critic_b0, critic_W1, critic_b1, pooled_machine):
    B = x.shape[0]
    f32 = jnp.float32
    feats = x[:, _OFF_FEATS:_OFF_ADJ].reshape(B, N, D)
    cand = x[:, _OFF_CAND:_OFF_MASK].astype(jnp.int32).reshape(B, N, 1)
    mask = x[:, _OFF_MASK:_ROW].reshape(B, N, 1)
    act3 = action.astype(jnp.int32).reshape(B, 1, 1)

    def row2(v):
        return v.reshape(1, -1).astype(f32)

    per_sample = lambda bs: pl.BlockSpec(bs, lambda b: (b,) + (0,) * (len(bs) - 1))
    shared = lambda arr: pl.BlockSpec(arr.shape, lambda b: (0,) * arr.ndim)

    weights = [enc_W0_0, row2(enc_b0_0), enc_W0_1, row2(enc_b0_1), enc_W0_2, row2(enc_b0_2),
               enc_W1_0, row2(enc_b1_0), enc_W1_1, row2(enc_b1_1), enc_W1_2, row2(enc_b1_2),
               actor_W0, row2(actor_b0), actor_W1, row2(actor_b1), actor_W2, row2(actor_b2),
               critic_W0, row2(critic_b0), critic_W1, row2(critic_b1), row2(pooled_machine)]

    in_specs = [pl.BlockSpec((1, 1, _ROW), lambda b: (b, 0, 0)),
                per_sample((1, N, D)),
                per_sample((1, N, 1)), per_sample((1, N, 1)),
                per_sample((1, 1, 1))] + [shared(w) for w in weights]

    out = pl.pallas_call(
        _fused,
        grid=(B,),
        in_specs=in_specs,
        out_specs=pl.BlockSpec((1, 1, 128), lambda b: (b, 0, 0)),
        out_shape=jax.ShapeDtypeStruct((B, 1, 128), f32),
        compiler_params=pltpu.CompilerParams(
            dimension_semantics=("parallel",),
            vmem_limit_bytes=120 * 1024 * 1024),
    )(x.reshape(B, 1, _ROW), feats, cand, mask, act3, *weights)

    return action, out[:, 0, 0], out[:, 0, 1], out[:, 0, 2:3]
